# baseline (speedup 1.0000x reference)
import functools

import jax
import jax.numpy as jnp
from jax import lax
from jax.experimental import pallas as pl
from jax.experimental.pallas import tpu as pltpu
from jax.experimental.pallas import tpu_sc as plsc

N = 1000000
C = 64
B = 16384
H = N // 2

_info = plsc.get_sparse_core_info()
_NC, _NS = _info.num_cores, _info.num_subcores
_NW = _NC * _NS
_B_PER_W = B // _NW
_SCH = 256

_mesh = plsc.VectorSubcoreMesh(core_axis_name="c", subcore_axis_name="s")


@functools.partial(
    pl.kernel,
    mesh=_mesh,
    out_type=jax.ShapeDtypeStruct((B, C), jnp.float32),
    compiler_params=pltpu.CompilerParams(
        use_tc_tiling_on_sc=False, needs_layout_passes=False
    ),
    scratch_types=[
        pltpu.VMEM((_B_PER_W,), jnp.int32),
        pltpu.VMEM((_B_PER_W,), jnp.int32),
        pltpu.VMEM((_SCH, C), jnp.float32),
        pltpu.VMEM((_SCH, C), jnp.float32),
        pltpu.VMEM((_SCH, C), jnp.float32),
        pltpu.SemaphoreType.DMA,
    ],
)
def _gather2_kernel(top_hbm, bot_hbm, idx_hbm, out_hbm,
                    idx_v, gidx_v, rows_t, rows_b, row_v, sem):
    wid = lax.axis_index("s") * _NC + lax.axis_index("c")
    base = wid * _B_PER_W
    pltpu.sync_copy(idx_hbm.at[pl.ds(base, _B_PER_W)], idx_v)

    lanes = lax.iota(jnp.int32, 16)

    def midx_body(t, _):
        j = t * 16 + lanes
        idxs = plsc.load_gather(idx_v, [j])
        m = idxs - jnp.where(idxs >= H, H, 0).astype(jnp.int32)
        plsc.store_scatter(gidx_v, [j], m)
        return 0

    lax.fori_loop(0, _B_PER_W // 16, midx_body, 0, unroll=2)

    for h in range(_B_PER_W // _SCH):
        hoff = h * _SCH
        pltpu.async_copy(
            top_hbm.at[gidx_v.at[pl.ds(hoff, _SCH)]], rows_t, sem
        )
        pltpu.async_copy(
            bot_hbm.at[gidx_v.at[pl.ds(hoff, _SCH)]], rows_b, sem
        )
        pltpu.make_async_copy(
            top_hbm.at[pl.ds(0, _SCH)], rows_t, sem
        ).wait()
        pltpu.make_async_copy(
            bot_hbm.at[pl.ds(0, _SCH)], rows_b, sem
        ).wait()

        def sel_body(t, _):
            j = hoff + t * 16 + lanes
            idxs = plsc.load_gather(idx_v, [j])
            is_bot = idxs >= H
            jl = t * 16 + lanes

            def col_body(c, _):
                csplat = jnp.full((16,), 0, jnp.int32) + c
                v_t = plsc.load_gather(rows_t, [jl, csplat])
                v_b = plsc.load_gather(rows_b, [jl, csplat])
                vals = jnp.where(is_bot, v_b, v_t)
                plsc.store_scatter(row_v, [jl, csplat], vals)
                return 0

            lax.fori_loop(0, C, col_body, 0, unroll=4)
            return 0

        lax.fori_loop(0, _SCH // 16, sel_body, 0)
        pltpu.sync_copy(row_v, out_hbm.at[pl.ds(base + hoff, _SCH)])


def kernel(logitsbank, index):
    top = lax.slice(logitsbank, (0, 0), (H, C))
    bot = lax.slice(logitsbank, (H, 0), (N, C))
    return _gather2_kernel(top, bot, index)


# speedup vs baseline: 2.6626x; 2.6626x over previous
import functools

import jax
import jax.numpy as jnp
from jax import lax
from jax.experimental import pallas as pl
from jax.experimental.pallas import tpu as pltpu
from jax.experimental.pallas import tpu_sc as plsc

N = 1000000
C = 64
B = 16384

_info = plsc.get_sparse_core_info()
_NC, _NS = _info.num_cores, _info.num_subcores
_NW = _NC * _NS
_B_PER_W = B // _NW
_NSEM = 4

_mesh = plsc.VectorSubcoreMesh(core_axis_name="c", subcore_axis_name="s")


@functools.partial(
    pl.kernel,
    mesh=_mesh,
    out_type=jax.ShapeDtypeStruct((B, C), jnp.float32),
    compiler_params=pltpu.CompilerParams(needs_layout_passes=False),
    scratch_types=[
        pltpu.VMEM((_B_PER_W,), jnp.int32),
        pltpu.VMEM((_B_PER_W, C), jnp.float32),
        pltpu.SemaphoreType.DMA,
        pltpu.SemaphoreType.DMA,
        pltpu.SemaphoreType.DMA,
        pltpu.SemaphoreType.DMA,
    ],
)
def _gather_kernel(bank_hbm, idx_hbm, out_hbm, idx_v, rows_v,
                   sem0, sem1, sem2, sem3):
    wid = lax.axis_index("s") * _NC + lax.axis_index("c")
    base = wid * _B_PER_W
    pltpu.sync_copy(idx_hbm.at[pl.ds(base, _B_PER_W)], idx_v)

    sems = [sem0, sem1, sem2, sem3]
    lanes = lax.iota(jnp.int32, 16)

    def fire_group(g, _):
        idxs = plsc.load_gather(idx_v, [g * 16 + lanes])
        for k in range(16):
            r = idxs[k]
            pltpu.async_copy(
                bank_hbm.at[pl.ds(r, 1)],
                rows_v.at[pl.ds(g * 16 + k, 1)],
                sems[k % _NSEM],
            )
        return 0

    lax.fori_loop(0, _B_PER_W // 16, fire_group, 0)
    q = _B_PER_W // _NSEM
    for i in range(_NSEM):
        pltpu.make_async_copy(
            bank_hbm.at[pl.ds(0, q)], rows_v.at[pl.ds(i * q, q)], sems[i]
        ).wait()
    pltpu.sync_copy(rows_v, out_hbm.at[pl.ds(base, _B_PER_W)])


def kernel(logitsbank, index):
    return _gather_kernel(logitsbank, index)
